# baseline (speedup 1.0000x reference)
import functools

import jax
import jax.numpy as jnp
from jax import lax
from jax.experimental import pallas as pl
from jax.experimental.pallas import tpu as pltpu
from jax.experimental.pallas import tpu_sc as plsc

_B = 16
_MAX_LEN = 4096
_D = 1024
_HALF = _D // 2
_NV = _HALF // 16
_R = 64


def _sc_mean(x, lengths):
    mesh = plsc.VectorSubcoreMesh(core_axis_name="c", subcore_axis_name="s")

    @functools.partial(
        pl.kernel,
        out_type=jax.ShapeDtypeStruct((_B, _D), jnp.float32),
        mesh=mesh,
        scratch_types=[
            pltpu.VMEM((32,), jnp.int32),
            pltpu.VMEM((_R, _HALF), jnp.float32),
            pltpu.VMEM((_R, _HALF), jnp.float32),
            pltpu.VMEM((_HALF,), jnp.float32),
            pltpu.SemaphoreType.DMA,
            pltpu.SemaphoreType.DMA,
        ],
    )
    def run(x_hbm, len_hbm, out_hbm, len_v, buf0, buf1, stage, sem0, sem1):
        wid = lax.axis_index("s") * 2 + lax.axis_index("c")
        b = wid // 2
        h = wid % 2
        col0 = h * _HALF

        pltpu.sync_copy(len_hbm, len_v.at[pl.ds(0, 16)])
        length = len_v[pl.ds(b, 16)][0]

        nchunks = (length + (_R - 1)) // _R
        npairs = (nchunks + 1) // 2

        def src(g):
            t0 = jnp.minimum(g * _R, _MAX_LEN - _R)
            return x_hbm.at[b, pl.ds(t0, _R), pl.ds(col0, _HALF)]

        pltpu.async_copy(src(0), buf0, sem0)
        pltpu.async_copy(src(1), buf1, sem1)

        def accum(buf, t0, acc):
            nr = jnp.clip(length - t0, 0, _R)

            def row(r, a):
                return tuple(a[j] + buf[r, pl.ds(16 * j, 16)]
                             for j in range(_NV))

            return lax.fori_loop(0, nr, row, acc)

        def pair(p, acc):
            g0 = 2 * p
            pltpu.make_async_copy(src(g0), buf0, sem0).wait()
            acc = accum(buf0, g0 * _R, acc)

            @pl.when(p + 1 < npairs)
            def _issue0():
                pltpu.async_copy(src(g0 + 2), buf0, sem0)

            pltpu.make_async_copy(src(g0 + 1), buf1, sem1).wait()
            acc = accum(buf1, (g0 + 1) * _R, acc)

            @pl.when(p + 1 < npairs)
            def _issue1():
                pltpu.async_copy(src(g0 + 3), buf1, sem1)

            return acc

        zero = jnp.zeros((16,), jnp.float32)
        acc = lax.fori_loop(0, npairs, pair, tuple(zero for _ in range(_NV)))

        rcp = jnp.ones((16,), jnp.float32) / length.astype(jnp.float32)
        for j in range(_NV):
            stage[pl.ds(16 * j, 16)] = acc[j] * rcp
        pltpu.sync_copy(stage, out_hbm.at[b, pl.ds(col0, _HALF)])

    return run(x, lengths)


def kernel(input_sequences, sequence_lengths):
    return _sc_mean(input_sequences, sequence_lengths.astype(jnp.int32))


# speedup vs baseline: 1.3666x; 1.3666x over previous
import functools

import jax
import jax.numpy as jnp
from jax import lax
from jax.experimental import pallas as pl
from jax.experimental.pallas import tpu as pltpu
from jax.experimental.pallas import tpu_sc as plsc

_B = 16
_MAX_LEN = 4096
_D = 1024
_HALF = _D // 2
_NV = _HALF // 16
_R = 64
_NRANGE = 16


def _zero_vec():
    return jnp.zeros((16,), jnp.float32)


def _mesh():
    return plsc.VectorSubcoreMesh(core_axis_name="c", subcore_axis_name="s")


def _partial_sums(x, lengths):
    @functools.partial(
        pl.kernel,
        out_type=jax.ShapeDtypeStruct((_NRANGE, _B, _D), jnp.float32),
        mesh=_mesh(),
        scratch_types=[
            pltpu.VMEM((32,), jnp.int32),
            pltpu.SMEM((16,), jnp.int32),
            pltpu.VMEM((_R, _HALF), jnp.float32),
            pltpu.VMEM((_R, _HALF), jnp.float32),
            pltpu.VMEM((_B, _HALF), jnp.float32),
            pltpu.SemaphoreType.DMA,
            pltpu.SemaphoreType.DMA,
        ],
    )
    def run(x_hbm, len_hbm, part_hbm, len_v, starts_s, buf0, buf1, stage,
            sem0, sem1):
        wid = lax.axis_index("s") * 2 + lax.axis_index("c")
        r = wid // 2
        h = wid % 2
        col0 = h * _HALF

        pltpu.sync_copy(len_hbm, len_v.at[pl.ds(0, 16)])

        total = jnp.int32(0)
        for b in range(_B):
            starts_s[b] = total
            total = total + len_v[pl.ds(b, 16)][0]

        lo = lax.shift_right_arithmetic(r * total, 4)
        hi = lax.shift_right_arithmetic((r + 1) * total, 4)

        zero = _zero_vec()

        def seq_body(b, carry):
            start = starts_s[b]
            lb = len_v[pl.ds(b, 16)][0]
            t_lo = jnp.clip(lo - start, 0, lb)
            t_hi = jnp.clip(hi - start, 0, lb)
            nrows = t_hi - t_lo

            for j in range(_NV):
                stage[b, pl.ds(16 * j, 16)] = zero

            @pl.when(nrows > 0)
            def _process():
                a_lo = t_lo & (-8)
                nch = lax.shift_right_arithmetic(t_hi - a_lo + (_R - 1), 6)
                npairs = lax.shift_right_arithmetic(nch + 1, 1)

                def src(g):
                    t0 = pl.multiple_of(
                        jnp.minimum(a_lo + g * _R, _MAX_LEN - _R), 8)
                    return x_hbm.at[b, pl.ds(t0, _R), pl.ds(col0, _HALF)]

                pltpu.async_copy(src(0), buf0, sem0)
                pltpu.async_copy(src(1), buf1, sem1)

                def accum(buf, g, acc):
                    gstart = a_lo + g * _R
                    t0 = jnp.minimum(gstart, _MAX_LEN - _R)
                    k_lo = jnp.maximum(t_lo, gstart) - t0
                    k_hi = jnp.minimum(t_hi, gstart + _R) - t0

                    def row(k, a):
                        return tuple(a[j] + buf[k, pl.ds(16 * j, 16)]
                                     for j in range(_NV))

                    return lax.fori_loop(k_lo, k_hi, row, acc)

                def pair(p, acc):
                    g0 = 2 * p
                    pltpu.make_async_copy(src(g0), buf0, sem0).wait()
                    acc = accum(buf0, g0, acc)

                    @pl.when(p + 1 < npairs)
                    def _issue0():
                        pltpu.async_copy(src(g0 + 2), buf0, sem0)

                    pltpu.make_async_copy(src(g0 + 1), buf1, sem1).wait()
                    acc = accum(buf1, g0 + 1, acc)

                    @pl.when(p + 1 < npairs)
                    def _issue1():
                        pltpu.async_copy(src(g0 + 3), buf1, sem1)

                    return acc

                acc = lax.fori_loop(0, npairs, pair,
                                    tuple(zero for _ in range(_NV)))
                for j in range(_NV):
                    stage[b, pl.ds(16 * j, 16)] = acc[j]

            return carry

        lax.fori_loop(0, _B, seq_body, jnp.int32(0))
        pltpu.sync_copy(stage, part_hbm.at[r, pl.ds(0, _B), pl.ds(col0, _HALF)])

    return run(x, lengths)


def _combine(partials, lengths):
    @functools.partial(
        pl.kernel,
        out_type=jax.ShapeDtypeStruct((_B, _D), jnp.float32),
        mesh=_mesh(),
        scratch_types=[
            pltpu.VMEM((32,), jnp.int32),
            pltpu.VMEM((_NRANGE, 8, 128), jnp.float32),
            pltpu.VMEM((8, 128), jnp.float32),
        ],
    )
    def run(part_hbm, len_hbm, out_hbm, len_v, buf, stage):
        wid = lax.axis_index("s") * 2 + lax.axis_index("c")

        @pl.when(wid < 16)
        def _active():
            g = wid // 8
            e = wid % 8
            row0 = 8 * g
            col0 = 128 * e

            pltpu.sync_copy(len_hbm, len_v.at[pl.ds(0, 16)])
            pltpu.sync_copy(
                part_hbm.at[pl.ds(0, _NRANGE), pl.ds(row0, 8),
                            pl.ds(col0, 128)],
                buf)

            for s in range(8):
                length = len_v[pl.ds(row0 + s, 16)][0]
                rcp = (jnp.ones((16,), jnp.float32)
                       / length.astype(jnp.float32))
                for j in range(8):
                    acc = _zero_vec()
                    for k in range(_NRANGE):
                        acc = acc + buf[k, s, pl.ds(16 * j, 16)]
                    stage[s, pl.ds(16 * j, 16)] = acc * rcp
            pltpu.sync_copy(
                stage, out_hbm.at[pl.ds(row0, 8), pl.ds(col0, 128)])

    return run(partials, lengths)


def kernel(input_sequences, sequence_lengths):
    lengths = sequence_lengths.astype(jnp.int32)
    partials = _partial_sums(input_sequences, lengths)
    return _combine(partials, lengths)


# speedup vs baseline: 1.4442x; 1.0569x over previous
import functools

import jax
import jax.numpy as jnp
from jax import lax
from jax.experimental import pallas as pl
from jax.experimental.pallas import tpu as pltpu
from jax.experimental.pallas import tpu_sc as plsc

_B = 16
_MAX_LEN = 4096
_D = 1024
_HALF = _D // 2
_NV = _HALF // 16
_R = 64
_NRANGE = 16


def _zero_vec():
    return jnp.zeros((16,), jnp.float32)


def _sc_mean(x, lengths):
    mesh = plsc.VectorSubcoreMesh(core_axis_name="c", subcore_axis_name="s")

    @functools.partial(
        pl.kernel,
        out_type=jax.ShapeDtypeStruct((_B, _D), jnp.float32),
        mesh=mesh,
        scratch_types=[
            pltpu.VMEM((32,), jnp.int32),
            pltpu.SMEM((16,), jnp.int32),
            pltpu.VMEM((_R, _HALF), jnp.float32),
            pltpu.VMEM((_R, _HALF), jnp.float32),
            pltpu.VMEM((_B, _HALF), jnp.float32),
            pltpu.VMEM_SHARED((_NRANGE, _B, _HALF), jnp.float32),
            pltpu.VMEM((_NRANGE, 8, 128), jnp.float32),
            pltpu.VMEM((8, 128), jnp.float32),
            pltpu.SemaphoreType.DMA,
            pltpu.SemaphoreType.DMA,
        ],
    )
    def run(x_hbm, len_hbm, out_hbm, len_v, starts_s, buf0, buf1, stage,
            shared, bufb, outb, sem0, sem1):
        c = lax.axis_index("c")
        s = lax.axis_index("s")
        col0 = c * _HALF

        pltpu.sync_copy(len_hbm, len_v.at[pl.ds(0, 16)])

        total = jnp.int32(0)
        for b in range(_B):
            starts_s[b] = total
            total = total + len_v[pl.ds(b, 16)][0]

        lo = lax.shift_right_arithmetic(s * total, 4)
        hi = lax.shift_right_arithmetic((s + 1) * total, 4)

        zero = _zero_vec()

        def seq_body(b, carry):
            start = starts_s[b]
            lb = len_v[pl.ds(b, 16)][0]
            t_lo = jnp.clip(lo - start, 0, lb)
            t_hi = jnp.clip(hi - start, 0, lb)
            nrows = t_hi - t_lo

            for j in range(_NV):
                stage[b, pl.ds(16 * j, 16)] = zero

            @pl.when(nrows > 0)
            def _process():
                a_lo = t_lo & (-8)
                nch = lax.shift_right_arithmetic(t_hi - a_lo + (_R - 1), 6)
                npairs = lax.shift_right_arithmetic(nch + 1, 1)

                def src(g):
                    t0 = pl.multiple_of(
                        jnp.minimum(a_lo + g * _R, _MAX_LEN - _R), 8)
                    return x_hbm.at[b, pl.ds(t0, _R), pl.ds(col0, _HALF)]

                pltpu.async_copy(src(0), buf0, sem0)
                pltpu.async_copy(src(1), buf1, sem1)

                def accum(buf, g, acc):
                    gstart = a_lo + g * _R
                    t0 = jnp.minimum(gstart, _MAX_LEN - _R)
                    k_lo = jnp.maximum(t_lo, gstart) - t0
                    k_hi = jnp.minimum(t_hi, gstart + _R) - t0

                    def row(k, a):
                        return tuple(a[j] + buf[k, pl.ds(16 * j, 16)]
                                     for j in range(_NV))

                    return lax.fori_loop(k_lo, k_hi, row, acc)

                def pair(p, acc):
                    g0 = 2 * p
                    pltpu.make_async_copy(src(g0), buf0, sem0).wait()
                    acc = accum(buf0, g0, acc)

                    @pl.when(p + 1 < npairs)
                    def _issue0():
                        pltpu.async_copy(src(g0 + 2), buf0, sem0)

                    pltpu.make_async_copy(src(g0 + 1), buf1, sem1).wait()
                    acc = accum(buf1, g0 + 1, acc)

                    @pl.when(p + 1 < npairs)
                    def _issue1():
                        pltpu.async_copy(src(g0 + 3), buf1, sem1)

                    return acc

                acc = lax.fori_loop(0, npairs, pair,
                                    tuple(zero for _ in range(_NV)))
                for j in range(_NV):
                    stage[b, pl.ds(16 * j, 16)] = acc[j]

            return carry

        lax.fori_loop(0, _B, seq_body, jnp.int32(0))

        pltpu.sync_copy(stage, shared.at[s])
        plsc.subcore_barrier()

        @pl.when(s < 8)
        def _combine():
            g = s // 4
            e = s % 4
            row0 = 8 * g
            cb = 128 * e

            pltpu.sync_copy(
                shared.at[pl.ds(0, _NRANGE), pl.ds(row0, 8), pl.ds(cb, 128)],
                bufb)

            for q in range(8):
                length = len_v[pl.ds(row0 + q, 16)][0]
                rcp = (jnp.ones((16,), jnp.float32)
                       / length.astype(jnp.float32))
                for j in range(8):
                    acc = _zero_vec()
                    for k in range(_NRANGE):
                        acc = acc + bufb[k, q, pl.ds(16 * j, 16)]
                    outb[q, pl.ds(16 * j, 16)] = acc * rcp
            pltpu.sync_copy(
                outb, out_hbm.at[pl.ds(row0, 8), pl.ds(col0 + cb, 128)])

    return run(x, lengths)


def kernel(input_sequences, sequence_lengths):
    return _sc_mean(input_sequences, sequence_lengths.astype(jnp.int32))
